# Initial kernel scaffold; baseline (speedup 1.0000x reference)
import jax
import jax.numpy as jnp
from jax.experimental import pallas as pl


def kernel(x_author, x_paper, edge_index_ap, edge_index_pa, params):
    raise NotImplementedError("write your pallas kernel here")


# speedup vs baseline: 45.4381x; 45.4381x over previous
import jax
import jax.numpy as jnp
from jax import lax
from jax.experimental import pallas as pl
from jax.experimental.pallas import tpu as pltpu
from jax.experimental.pallas import tpu_sc as plsc

N = 10000
HID = 256
HEADS = 8
DHEAD = 32
OUT = 10
E = 300000
NEG_SLOPE = 0.2
EPS = 1e-16

BN = 1000
NB = N // BN

C = 128
TILES = 16
CHUNKS_PER_TILE = 147
EPT = CHUNKS_PER_TILE * C
E_PAD = EPT * TILES

f32 = jnp.float32
i32 = jnp.int32

_SC_PARAMS = dict(
    compiler_params=pltpu.CompilerParams(
        needs_layout_passes=False, use_tc_tiling_on_sc=False),
)


def _pair_expand(s2):
    sel = (lax.broadcasted_iota(i32, (2, 64), 1) // DHEAD
           == lax.broadcasted_iota(i32, (2, 64), 0)).astype(f32)
    return jnp.dot(s2, sel, preferred_element_type=f32)


def _quarters_to_x(a_refs, s_refs):
    xs = []
    for q, (a_ref, s_ref) in enumerate(zip(a_refs, s_refs)):
        p = q % 2
        den = _pair_expand(s_ref[:, 2 * p:2 * p + 2]) + EPS
        xs.append(jnp.maximum(a_ref[...] / den, 0.0))
    return xs


def _proj_epilogue(g, nb, h, bs_ref, bd_ref, hlo_ref, hhi_ref, as_ref, ad_ref,
                   mbs_ref, mbd_ref):
    a_src = jnp.dot(h, bs_ref[0], preferred_element_type=f32)
    a_dst = jnp.dot(h, bd_ref[0], preferred_element_type=f32)
    hlo_ref[...] = h[:, :64]
    hhi_ref[...] = h[:, 64:]
    as_ref[...] = a_src
    ad_ref[...] = a_dst

    @pl.when((g == 0) & (nb == 0))
    def _():
        mbs_ref[...] = jnp.full((HEADS, 16), -1e30, f32)
        mbd_ref[...] = jnp.full((HEADS, 16), -1e30, f32)

    ms = jnp.broadcast_to(jnp.max(a_src, axis=0)[:, None], (4, 16))
    md = jnp.broadcast_to(jnp.max(a_dst, axis=0)[:, None], (4, 16))
    sl = pl.ds(4 * g, 4)
    mbs_ref[sl, :] = jnp.maximum(mbs_ref[sl, :], ms)
    mbd_ref[sl, :] = jnp.maximum(mbd_ref[sl, :], md)


def _proj_raw_body(x_ref, w_ref, b_ref, bs_ref, bd_ref,
                   hlo_ref, hhi_ref, as_ref, ad_ref, mbs_ref, mbd_ref):
    g, nb = pl.program_id(0), pl.program_id(1)
    h = jnp.dot(x_ref[...], w_ref[...], preferred_element_type=f32) + b_ref[0]
    _proj_epilogue(g, nb, h, bs_ref, bd_ref, hlo_ref, hhi_ref, as_ref, ad_ref,
                   mbs_ref, mbd_ref)


def _proj_agg_body(a0, a1, a2, a3, s0, s1, s2, s3, w_ref, b_ref, bs_ref,
                   bd_ref, hlo_ref, hhi_ref, as_ref, ad_ref, mbs_ref, mbd_ref):
    g, nb = pl.program_id(0), pl.program_id(1)
    xs = _quarters_to_x((a0, a1, a2, a3), (s0, s1, s2, s3))
    w = w_ref[...]
    h = b_ref[0]
    for q in range(4):
        h = h + jnp.dot(xs[q], w[64 * q:64 * (q + 1)],
                        preferred_element_type=f32)
    _proj_epilogue(g, nb, h, bs_ref, bd_ref, hlo_ref, hhi_ref, as_ref, ad_ref,
                   mbs_ref, mbd_ref)


def _proj_out_shapes():
    return (
        jax.ShapeDtypeStruct((2 * N, 64), f32),
        jax.ShapeDtypeStruct((2 * N, 64), f32),
        jax.ShapeDtypeStruct((2 * N, 4), f32),
        jax.ShapeDtypeStruct((2 * N, 4), f32),
        jax.ShapeDtypeStruct((HEADS, 16), f32),
        jax.ShapeDtypeStruct((HEADS, 16), f32),
    )


def _proj_out_specs():
    half = lambda g, nb: (g * NB + nb, 0)
    return (
        pl.BlockSpec((BN, 64), half),
        pl.BlockSpec((BN, 64), half),
        pl.BlockSpec((BN, 4), half),
        pl.BlockSpec((BN, 4), half),
        pl.BlockSpec((HEADS, 16), lambda g, nb: (0, 0)),
        pl.BlockSpec((HEADS, 16), lambda g, nb: (0, 0)),
    )


def _quarter_in_specs(nargs=2):
    specs = [pl.BlockSpec((BN, 64), lambda g, nb, q=q: (q * NB + nb, 0))
             for q in range(4)]
    specs += [pl.BlockSpec((BN, 16), lambda g, nb, c=q // 2: (c * NB + nb, 0))
              for q in range(4)]
    return specs


def _proj_raw(x, w, b2, bsrc, bdst):
    din = x.shape[1]
    return pl.pallas_call(
        _proj_raw_body,
        grid=(2, NB),
        in_specs=[
            pl.BlockSpec((BN, din), lambda g, nb: (nb, 0)),
            pl.BlockSpec((din, 128), lambda g, nb: (0, g)),
            pl.BlockSpec((1, 1, 128), lambda g, nb: (g, 0, 0)),
            pl.BlockSpec((1, 128, 4), lambda g, nb: (g, 0, 0)),
            pl.BlockSpec((1, 128, 4), lambda g, nb: (g, 0, 0)),
        ],
        out_specs=_proj_out_specs(),
        out_shape=_proj_out_shapes(),
    )(x, w, b2, bsrc, bdst)


def _proj_agg(agg, s4, w, b2, bsrc, bdst):
    return pl.pallas_call(
        _proj_agg_body,
        grid=(2, NB),
        in_specs=_quarter_in_specs() + [
            pl.BlockSpec((HID, 128), lambda g, nb: (0, g)),
            pl.BlockSpec((1, 1, 128), lambda g, nb: (g, 0, 0)),
            pl.BlockSpec((1, 128, 4), lambda g, nb: (g, 0, 0)),
            pl.BlockSpec((1, 128, 4), lambda g, nb: (g, 0, 0)),
        ],
        out_specs=_proj_out_specs(),
        out_shape=_proj_out_shapes(),
    )(agg, agg, agg, agg, s4, s4, s4, s4, w, b2, bsrc, bdst)


def _lin_body(a0, a1, a2, a3, s0, s1, s2, s3, w_ref, b_ref, o_ref):
    xs = _quarters_to_x((a0, a1, a2, a3), (s0, s1, s2, s3))
    acc = jnp.broadcast_to(b_ref[...], (BN, OUT))
    for q in range(4):
        acc = acc + jnp.dot(xs[q], w_ref[q], preferred_element_type=f32)
    o_ref[...] = acc


def _lin(agg, s4, w4, b):
    qspecs = [pl.BlockSpec((BN, 64), lambda nb, q=q: (q * NB + nb, 0))
              for q in range(4)]
    qspecs += [pl.BlockSpec((BN, 16), lambda nb, c=q // 2: (c * NB + nb, 0))
               for q in range(4)]
    return pl.pallas_call(
        _lin_body,
        grid=(NB,),
        in_specs=qspecs + [
            pl.BlockSpec((4, 64, OUT), lambda nb: (0, 0, 0)),
            pl.BlockSpec((1, OUT), lambda nb: (0, 0)),
        ],
        out_specs=pl.BlockSpec((BN, OUT), lambda nb: (nb, 0)),
        out_shape=jax.ShapeDtypeStruct((N, OUT), f32),
    )(agg, agg, agg, agg, s4, s4, s4, s4, w4, b)


def _edge_body(ed_hbm, asrc_hbm, adst_hbm, hlo_hbm, hhi_hbm, mb_hbm,
               agg_out,
               asrc_v, adst_v, ed_v, dst_v, srcg_v, rows_v,
               mb_v, agg_sh, sem):
    c = lax.axis_index("c")
    sid = lax.axis_index("s")
    iota = lax.iota(i32, 16)
    zeros16 = jnp.zeros((16,), f32)

    pltpu.sync_copy(asrc_hbm.at[pl.ds(c * 4 * N, 4 * N)], asrc_v)
    pltpu.sync_copy(adst_hbm.at[pl.ds(c * 4 * N, 4 * N)], adst_v)
    pltpu.sync_copy(mb_hbm.at[pl.ds(16 * c, 16)], mb_v)

    base_e = sid * EPT

    for p in range(2):
        q_off = (2 * c + p) * N
        htab = hlo_hbm if p == 0 else hhi_hbm

        def _zr(r, _):
            for o in (0, 16, 32, 48):
                rows_v[r, pl.ds(o, 16)] = zeros16
            return 0

        lax.fori_loop(0, C, _zr, 0)
        for m in range(8):
            k = sid * 8 + m

            @pl.when(k < 125)
            def _():
                pltpu.sync_copy(rows_v.at[pl.ds(0, 80)],
                                agg_sh.at[pl.ds(k * 80, 80)])

        plsc.subcore_barrier()

        m16 = mb_v[...]
        mb_vec = [jnp.broadcast_to(m16[hh], (16,))
                  for hh in (2 * p, 2 * p + 1)]

        def chunk(ci, _):
            off = base_e + ci * C
            pltpu.sync_copy(ed_hbm.at[pl.ds(off, C)], ed_v)

            def _adj(i, _):
                sl = pl.ds(i * 16, 16)
                packed = ed_v[sl]
                srcg_v[sl] = (packed >> 14) + c * N
                dst_v[sl] = packed & 16383
                return 0

            lax.fori_loop(0, C // 16, _adj, 0)
            pltpu.async_copy(htab.at[srcg_v], rows_v, sem).wait()

            def jstep(j, _):
                sl = pl.ds(j * 16, 16)
                src16 = srcg_v[sl] - c * N
                dst16 = dst_v[sl]
                valid = (off + j * 16 + iota) < E
                wvecs = []
                for hp, hh in enumerate((2 * p, 2 * p + 1)):
                    sa = plsc.load_gather(asrc_v, [src16 * 4 + hh])
                    sd = plsc.load_gather(adst_v, [dst16 * 4 + hh])
                    z = sa + sd
                    t = jnp.maximum(z, NEG_SLOPE * z)
                    wvecs.append(
                        jnp.where(valid, jnp.exp(t - mb_vec[hp]), 0.0))

                for e in range(16):
                    eg = j * 16 + e
                    for hp in range(2):
                        ws = jnp.broadcast_to(wvecs[hp][e], (16,))
                        for k in range(2):
                            fsl = pl.ds(32 * hp + 16 * k, 16)
                            rows_v[eg, fsl] = rows_v[eg, fsl] * ws
                return 0

            lax.fori_loop(0, C // 16, jstep, 0)

            pltpu.sync_copy(rows_v, agg_sh.at[dst_v], add=True)
            return 0

        lax.fori_loop(0, CHUNKS_PER_TILE, chunk, 0)
        plsc.subcore_barrier()

        for m in range(8):
            k = sid * 8 + m

            @pl.when(k < 125)
            def _():
                sl_sh = pl.ds(k * 80, 80)
                sl_out = pl.ds(q_off + k * 80, 80)
                pltpu.sync_copy(agg_sh.at[sl_sh], agg_out.at[sl_out])

        if p == 0:
            plsc.subcore_barrier()


def _s_body(ed_hbm, asrc_hbm, adst_hbm, mb_hbm, s_out,
            asrc_v, adst_v, ed_v, src_v, dst_v, wmsg_v, mb_v, s_sh):
    c = lax.axis_index("c")
    sid = lax.axis_index("s")
    iota = lax.iota(i32, 16)
    zeros16 = jnp.zeros((16,), f32)

    pltpu.sync_copy(asrc_hbm.at[pl.ds(c * 4 * N, 4 * N)], asrc_v)
    pltpu.sync_copy(adst_hbm.at[pl.ds(c * 4 * N, 4 * N)], adst_v)
    pltpu.sync_copy(mb_hbm.at[pl.ds(16 * c, 16)], mb_v)

    def _zw(r, _):
        wmsg_v[r, :] = zeros16
        return 0

    lax.fori_loop(0, C, _zw, 0)

    for m in range(8):
        k = sid * 8 + m

        @pl.when(k < 125)
        def _():
            pltpu.sync_copy(wmsg_v.at[pl.ds(0, 80)], s_sh.at[pl.ds(k * 80, 80)])

    plsc.subcore_barrier()

    m16 = mb_v[...]
    mb_vec = [jnp.broadcast_to(m16[hh], (16,)) for hh in range(4)]

    base_e = sid * EPT

    def chunk(ci, _):
        off = base_e + ci * C
        pltpu.sync_copy(ed_hbm.at[pl.ds(off, C)], ed_v)

        def _adj(i, _):
            sl = pl.ds(i * 16, 16)
            packed = ed_v[sl]
            src_v[sl] = packed >> 14
            dst_v[sl] = packed & 16383
            return 0

        lax.fori_loop(0, C // 16, _adj, 0)

        onehots = [(iota == hh).astype(f32) for hh in range(4)]

        def jstep(j, _):
            sl = pl.ds(j * 16, 16)
            src16 = src_v[sl]
            dst16 = dst_v[sl]
            valid = (off + j * 16 + iota) < E
            wvecs = []
            for hh in range(4):
                sa = plsc.load_gather(asrc_v, [src16 * 4 + hh])
                sd = plsc.load_gather(adst_v, [dst16 * 4 + hh])
                z = sa + sd
                t = jnp.maximum(z, NEG_SLOPE * z)
                wvecs.append(jnp.where(valid, jnp.exp(t - mb_vec[hh]), 0.0))

            for e in range(16):
                eg = j * 16 + e
                row = (wvecs[0][e] * onehots[0] + wvecs[1][e] * onehots[1]
                       + wvecs[2][e] * onehots[2] + wvecs[3][e] * onehots[3])
                wmsg_v[eg, :] = row
            return 0

        lax.fori_loop(0, C // 16, jstep, 0)
        pltpu.sync_copy(wmsg_v, s_sh.at[dst_v], add=True)
        return 0

    lax.fori_loop(0, CHUNKS_PER_TILE, chunk, 0)
    plsc.subcore_barrier()

    for m in range(8):
        k = sid * 8 + m

        @pl.when(k < 125)
        def _():
            sl_sh = pl.ds(k * 80, 80)
            sl_out = pl.ds(c * N + k * 80, 80)
            pltpu.sync_copy(s_sh.at[sl_sh], s_out.at[sl_out])


def _edge_call(ed, asrc2, adst2, hlo, hhi, mbs, mbd):
    mesh = plsc.VectorSubcoreMesh(core_axis_name="c", subcore_axis_name="s")
    zc = mbs[:, 0] + mbd[:, 0]
    bounds = jnp.maximum(zc, NEG_SLOPE * zc)
    pad12 = jnp.zeros((12,), f32)
    mbcat = jnp.concatenate([bounds[0:4], pad12, bounds[4:8], pad12])
    asrc_flat = asrc2.reshape(-1)
    adst_flat = adst2.reshape(-1)
    agg = pl.kernel(
        _edge_body,
        out_type=jax.ShapeDtypeStruct((4 * N, 64), f32),
        mesh=mesh,
        **_SC_PARAMS,
        scratch_types=[
            pltpu.VMEM((4 * N,), f32),
            pltpu.VMEM((4 * N,), f32),
            pltpu.VMEM((C,), i32),
            pltpu.VMEM((C,), i32),
            pltpu.VMEM((C,), i32),
            pltpu.VMEM((C, 64), f32),
            pltpu.VMEM((16,), f32),
            pltpu.VMEM_SHARED((N, 64), f32),
            pltpu.SemaphoreType.DMA,
        ],
    )(ed, asrc_flat, adst_flat, hlo, hhi, mbcat)
    s4 = pl.kernel(
        _s_body,
        out_type=jax.ShapeDtypeStruct((2 * N, 16), f32),
        mesh=mesh,
        **_SC_PARAMS,
        scratch_types=[
            pltpu.VMEM((4 * N,), f32),
            pltpu.VMEM((4 * N,), f32),
            pltpu.VMEM((C,), i32),
            pltpu.VMEM((C,), i32),
            pltpu.VMEM((C,), i32),
            pltpu.VMEM((C, 16), f32),
            pltpu.VMEM((16,), f32),
            pltpu.VMEM_SHARED((N, 16), f32),
        ],
    )(ed, asrc_flat, adst_flat, mbcat)
    return agg, s4


def _blockdiag(att):
    a4 = att.reshape(2, 4, DHEAD)
    eye = jnp.eye(4, dtype=f32)
    return (eye[None, :, None, :] * a4[:, :, :, None]).reshape(2, 128, 4)


def _pad_edges(ei):
    pad = E_PAD - E
    fill = (jnp.arange(pad, dtype=i32) * 97) % N
    src = jnp.concatenate([ei[0], fill])
    dst = jnp.concatenate([ei[1], fill])
    return src * 16384 + dst


def kernel(x_author, x_paper, edge_index_ap, edge_index_pa, params):
    ed_ap = _pad_edges(edge_index_ap)
    ed_pa = _pad_edges(edge_index_pa)

    state = {"author": ("raw", x_author, None), "paper": ("raw", x_paper, None)}
    for li in range(3):
        proj = {}
        for nt, et_src, et_dst in (("author", "ap", "pa"), ("paper", "pa", "ap")):
            w = params[f"l{li}_proj_{nt}_W"]
            b2 = params[f"l{li}_proj_{nt}_b"].reshape(2, 1, 128)
            bsrc = _blockdiag(params[f"l{li}_att_src_{et_src}"])
            bdst = _blockdiag(params[f"l{li}_att_dst_{et_dst}"])
            kind, a, s4 = state[nt]
            if kind == "raw":
                proj[nt] = _proj_raw(a, w, b2, bsrc, bdst)
            else:
                proj[nt] = _proj_agg(a, s4, w, b2, bsrc, bdst)
        hlo_a, hhi_a, asrc_ap, adst_pa, mbs_ap, mbd_pa = proj["author"]
        hlo_p, hhi_p, asrc_pa, adst_ap, mbs_pa, mbd_ap = proj["paper"]
        agg_p, s4_p = _edge_call(ed_ap, asrc_ap, adst_ap, hlo_a, hhi_a,
                                 mbs_ap, mbd_ap)
        agg_a, s4_a = _edge_call(ed_pa, asrc_pa, adst_pa, hlo_p, hhi_p,
                                 mbs_pa, mbd_pa)
        state = {"author": ("agg", agg_a, s4_a), "paper": ("agg", agg_p, s4_p)}

    _, agg_a, s4_a = state["author"]
    w4 = params["lin_W"].reshape(4, 64, OUT)
    b = params["lin_b"].reshape(1, OUT)
    return _lin(agg_a, s4_a, w4, b)
